# baseline (device time: 268878 ns/iter reference)
import jax
import jax.numpy as jnp
from jax import lax
from jax.experimental import pallas as pl
from jax.experimental.pallas import tpu as pltpu

N_DEV = 16
B, SQ, SKV = 2, 512, 512
HQ_PER, DH = 8, 64
D_MODEL = 768
ROWS = B * SQ
CHUNK = ROWS // N_DEV
N_STEPS = 2 * (N_DEV - 1)


HQ_TOT = 128
SBLK = 64


def _select_heads(arr):
    def sel_body(a_ref, o_ref):
        my_pos = lax.axis_index("i")
        hl = lax.broadcasted_iota(jnp.int32, (HQ_PER, HQ_TOT), 0)
        hg = lax.broadcasted_iota(jnp.int32, (HQ_PER, HQ_TOT), 1)
        onehot = (hg == hl + my_pos * HQ_PER).astype(jnp.float32)
        o_ref[...] = lax.dot_general(
            onehot, a_ref[...], (((1,), (2,)), ((), ())),
            preferred_element_type=jnp.float32)

    return pl.pallas_call(
        sel_body,
        grid=(SKV // SBLK,),
        in_specs=[pl.BlockSpec((B, SBLK, HQ_TOT, DH), lambda i: (0, i, 0, 0))],
        out_specs=pl.BlockSpec((HQ_PER, B, SBLK, DH), lambda i: (0, 0, i, 0)),
        out_shape=jax.ShapeDtypeStruct((HQ_PER, B, SKV, DH), jnp.float32),
    )(arr)


def kernel(x, Wq, K_ext, V_ext, Wo):
    K_sh = _select_heads(K_ext)
    V_sh = _select_heads(V_ext)

    def body(x_ref, wq_ref, k_ref, v_ref, wo_ref, out_ref,
             ctx_ref, sbuf_ref, comm_ref, send_sems, recv_sems, credit_sem):
        my_pos = lax.axis_index("i")
        left = lax.rem(my_pos + N_DEV - 1, N_DEV)
        right = lax.rem(my_pos + 1, N_DEV)

        barrier_sem = pltpu.get_barrier_semaphore()
        for nbr in (left, right):
            pl.semaphore_signal(barrier_sem, inc=1, device_id=(nbr,),
                                device_id_type=pl.DeviceIdType.MESH)
        pl.semaphore_wait(barrier_sem, 2)

        qi = lax.broadcasted_iota(jnp.int32, (SQ, SKV), 0)
        ki = lax.broadcasted_iota(jnp.int32, (SQ, SKV), 1)
        mask = (jnp.abs(qi - ki) <= 128) | (ki < 32) | (qi < 32)
        neg = jnp.float32(-1e9)

        for b in range(B):
            q_all = jnp.dot(x_ref[b, :, :], wq_ref[:, :],
                            preferred_element_type=jnp.float32)
            for h in range(HQ_PER):
                q_h = q_all[:, h * DH:(h + 1) * DH]
                k_h = k_ref[h, b, :, :]
                v_h = v_ref[h, b, :, :]
                s = lax.dot_general(q_h, k_h, (((1,), (1,)), ((), ())),
                                    preferred_element_type=jnp.float32)
                s = s * jnp.float32(0.125)
                s = jnp.where(mask, s, neg)
                m = jnp.max(s, axis=1, keepdims=True)
                w = jnp.exp(s - m)
                w = w / jnp.sum(w, axis=1, keepdims=True)
                ctx_ref[:, h * DH:(h + 1) * DH] = jnp.dot(
                    w, v_h, preferred_element_type=jnp.float32)
            out_ref[b * SQ:(b + 1) * SQ, :] = jnp.dot(
                ctx_ref[:, :], wo_ref[:, :],
                preferred_element_type=jnp.float32)

        for s_i in range(N_STEPS):
            slot = s_i % 2
            if s_i >= 2:
                pl.semaphore_wait(credit_sem, 1)
            if s_i < N_DEV - 1:
                sc = lax.rem(my_pos - s_i + 2 * N_DEV, N_DEV)
                rc = lax.rem(my_pos - s_i - 1 + 2 * N_DEV, N_DEV)
            else:
                ag = s_i - (N_DEV - 1)
                sc = lax.rem(my_pos + 1 - ag + 2 * N_DEV, N_DEV)
                rc = lax.rem(my_pos - ag + 2 * N_DEV, N_DEV)
            sbuf_ref[slot] = out_ref[
                pl.ds(sc * CHUNK, CHUNK), :].astype(jnp.bfloat16)
            rdma = pltpu.make_async_remote_copy(
                src_ref=sbuf_ref.at[slot],
                dst_ref=comm_ref.at[slot],
                send_sem=send_sems.at[slot],
                recv_sem=recv_sems.at[slot],
                device_id=(right,),
                device_id_type=pl.DeviceIdType.MESH,
            )
            rdma.start()
            rdma.wait()
            inc = comm_ref[slot].astype(jnp.float32)
            if s_i < N_DEV - 1:
                acc = out_ref[pl.ds(rc * CHUNK, CHUNK), :]
                out_ref[pl.ds(rc * CHUNK, CHUNK), :] = acc + inc
            else:
                out_ref[pl.ds(rc * CHUNK, CHUNK), :] = inc
            if s_i < N_STEPS - 2:
                pl.semaphore_signal(credit_sem, inc=1, device_id=(left,),
                                    device_id_type=pl.DeviceIdType.MESH)

    out = pl.pallas_call(
        body,
        out_shape=jax.ShapeDtypeStruct((ROWS, D_MODEL), jnp.float32),
        in_specs=[pl.BlockSpec(memory_space=pltpu.VMEM)] * 5,
        out_specs=pl.BlockSpec(memory_space=pltpu.VMEM),
        scratch_shapes=[
            pltpu.VMEM((SQ, HQ_PER * DH), jnp.float32),
            pltpu.VMEM((2, CHUNK, D_MODEL), jnp.bfloat16),
            pltpu.VMEM((2, CHUNK, D_MODEL), jnp.bfloat16),
            pltpu.SemaphoreType.DMA((2,)),
            pltpu.SemaphoreType.DMA((2,)),
            pltpu.SemaphoreType.REGULAR,
        ],
        compiler_params=pltpu.CompilerParams(collective_id=0),
    )(x, Wq, K_sh, V_sh, Wo)
    return out.reshape(B, SQ, D_MODEL)


# device time: 152703 ns/iter; 1.7608x vs baseline; 1.7608x over previous
import jax
import jax.numpy as jnp
from jax import lax
from jax.experimental import pallas as pl
from jax.experimental.pallas import tpu as pltpu

N_DEV = 16
B, SQ, SKV = 2, 512, 512
HQ_PER, DH = 8, 64
D_MODEL = 768
ROWS = B * SQ
CHUNK = ROWS // N_DEV
N_STEPS = 2 * (N_DEV - 1)


def kernel(x, Wq, K_ext, V_ext, Wo):
    my = lax.axis_index("i")
    K_sh = lax.dynamic_slice_in_dim(K_ext, my * HQ_PER, HQ_PER, axis=2)
    V_sh = lax.dynamic_slice_in_dim(V_ext, my * HQ_PER, HQ_PER, axis=2)

    def body(x_ref, wq_ref, k_ref, v_ref, wo_ref, out_ref,
             ctx_ref, sbuf_ref, comm_ref, send_sems, recv_sems):
        my_pos = lax.axis_index("i")

        barrier_sem = pltpu.get_barrier_semaphore()
        for d in (1, 2, 4, 8):
            pl.semaphore_signal(barrier_sem, inc=1,
                                device_id=(my_pos ^ d,),
                                device_id_type=pl.DeviceIdType.MESH)
        pl.semaphore_wait(barrier_sem, 4)

        qi = lax.broadcasted_iota(jnp.int32, (SQ, SKV), 0)
        ki = lax.broadcasted_iota(jnp.int32, (SQ, SKV), 1)
        mask = (jnp.abs(qi - ki) <= 128) | (ki < 32) | (qi < 32)
        neg = jnp.float32(-1e9)

        for b in range(B):
            q_all = jnp.dot(x_ref[b, :, :], wq_ref[:, :],
                            preferred_element_type=jnp.float32)
            for h in range(HQ_PER):
                q_h = q_all[:, h * DH:(h + 1) * DH]
                k_h = k_ref[b, :, h, :]
                v_h = v_ref[b, :, h, :]
                s = lax.dot_general(q_h, k_h, (((1,), (1,)), ((), ())),
                                    preferred_element_type=jnp.float32)
                s = s * jnp.float32(0.125)
                s = jnp.where(mask, s, neg)
                m = jnp.max(s, axis=1, keepdims=True)
                w = jnp.exp(s - m)
                w = w / jnp.sum(w, axis=1, keepdims=True)
                ctx_ref[:, h * DH:(h + 1) * DH] = jnp.dot(
                    w, v_h, preferred_element_type=jnp.float32)
            out_ref[b * SQ:(b + 1) * SQ, :] = jnp.dot(
                ctx_ref[:, :], wo_ref[:, :],
                preferred_element_type=jnp.float32)

        seg_start = jnp.int32(0)

        half = ROWS
        for slot, d in enumerate((1, 2, 4, 8)):
            half //= 2
            partner = my_pos ^ d
            has_bit = (my_pos & d) > 0
            send_off = pl.multiple_of(
                jnp.where(has_bit, seg_start, seg_start + half), CHUNK)
            keep_off = pl.multiple_of(
                jnp.where(has_bit, seg_start + half, seg_start), CHUNK)
            sbuf_ref[0:half, :] = out_ref[
                pl.ds(send_off, half), :].astype(jnp.bfloat16)
            rdma = pltpu.make_async_remote_copy(
                src_ref=sbuf_ref.at[0:half, :],
                dst_ref=comm_ref.at[slot, 0:half, :],
                send_sem=send_sems.at[slot],
                recv_sem=recv_sems.at[slot],
                device_id=(partner,),
                device_id_type=pl.DeviceIdType.MESH,
            )
            rdma.start()
            rdma.wait()
            acc = out_ref[pl.ds(keep_off, half), :]
            out_ref[pl.ds(keep_off, half), :] = (
                acc + comm_ref[slot, 0:half, :].astype(jnp.float32))
            seg_start = keep_off

        seg_len = ROWS // N_DEV
        for slot, d in enumerate((8, 4, 2, 1)):
            slot += 4
            partner = my_pos ^ d
            rowbit = (ROWS // 2) // d
            seg_start = pl.multiple_of(seg_start, CHUNK)
            sbuf_ref[0:seg_len, :] = out_ref[
                pl.ds(seg_start, seg_len), :].astype(jnp.bfloat16)
            rdma = pltpu.make_async_remote_copy(
                src_ref=sbuf_ref.at[0:seg_len, :],
                dst_ref=comm_ref.at[slot, 0:seg_len, :],
                send_sem=send_sems.at[slot],
                recv_sem=recv_sems.at[slot],
                device_id=(partner,),
                device_id_type=pl.DeviceIdType.MESH,
            )
            rdma.start()
            rdma.wait()
            partner_start = pl.multiple_of(seg_start ^ rowbit, CHUNK)
            out_ref[pl.ds(partner_start, seg_len), :] = comm_ref[
                slot, 0:seg_len, :].astype(jnp.float32)
            seg_start = jnp.bitwise_and(seg_start, jnp.int32(~rowbit))
            seg_len *= 2

    out = pl.pallas_call(
        body,
        out_shape=jax.ShapeDtypeStruct((ROWS, D_MODEL), jnp.float32),
        in_specs=[pl.BlockSpec(memory_space=pltpu.VMEM)] * 5,
        out_specs=pl.BlockSpec(memory_space=pltpu.VMEM),
        scratch_shapes=[
            pltpu.VMEM((SQ, HQ_PER * DH), jnp.float32),
            pltpu.VMEM((ROWS // 2, D_MODEL), jnp.bfloat16),
            pltpu.VMEM((8, ROWS // 2, D_MODEL), jnp.bfloat16),
            pltpu.SemaphoreType.DMA((8,)),
            pltpu.SemaphoreType.DMA((8,)),
        ],
        compiler_params=pltpu.CompilerParams(collective_id=0),
    )(x, Wq, K_sh, V_sh, Wo)
    return out.reshape(B, SQ, D_MODEL)


# device time: 123089 ns/iter; 2.1844x vs baseline; 1.2406x over previous
import jax
import jax.numpy as jnp
from jax import lax
from jax.experimental import pallas as pl
from jax.experimental.pallas import tpu as pltpu

N_DEV = 16
B, SQ, SKV = 2, 512, 512
HQ_PER, DH = 8, 64
D_MODEL = 768
ROWS = B * SQ
CHUNK = ROWS // N_DEV
N_STEPS = 2 * (N_DEV - 1)


def kernel(x, Wq, K_ext, V_ext, Wo):
    my = lax.axis_index("i")
    K_sh = lax.dynamic_slice_in_dim(
        K_ext.astype(jnp.bfloat16), my * HQ_PER, HQ_PER, axis=2)
    V_sh = lax.dynamic_slice_in_dim(
        V_ext.astype(jnp.bfloat16), my * HQ_PER, HQ_PER, axis=2)
    x = x.astype(jnp.bfloat16)
    Wq = Wq.astype(jnp.bfloat16)
    Wo = Wo.astype(jnp.bfloat16)

    def body(x_ref, wq_ref, k_ref, v_ref, wo_ref, out_ref,
             ctx_ref, sbuf_ref, comm_ref, send_sems, recv_sems):
        my_pos = lax.axis_index("i")

        barrier_sem = pltpu.get_barrier_semaphore()
        for d in (1, 2, 4, 8):
            pl.semaphore_signal(barrier_sem, inc=1,
                                device_id=(my_pos ^ d,),
                                device_id_type=pl.DeviceIdType.MESH)
        pl.semaphore_wait(barrier_sem, 4)

        qi = lax.broadcasted_iota(jnp.int32, (SQ, SKV), 0)
        ki = lax.broadcasted_iota(jnp.int32, (SQ, SKV), 1)
        mask = (jnp.abs(qi - ki) <= 128) | (ki < 32) | (qi < 32)
        neg = jnp.float32(-1e9)

        for b in range(B):
            q_all = jnp.dot(x_ref[b, :, :], wq_ref[:, :],
                            preferred_element_type=jnp.float32)
            q_all = q_all.astype(jnp.bfloat16)
            for h in range(HQ_PER):
                q_h = q_all[:, h * DH:(h + 1) * DH]
                k_h = k_ref[b, :, h, :]
                v_h = v_ref[b, :, h, :]
                s = lax.dot_general(q_h, k_h, (((1,), (1,)), ((), ())),
                                    preferred_element_type=jnp.float32)
                s = s * jnp.float32(0.125)
                s = jnp.where(mask, s, neg)
                m = jnp.max(s, axis=1, keepdims=True)
                w = jnp.exp(s - m)
                w = w / jnp.sum(w, axis=1, keepdims=True)
                ctx_ref[:, h * DH:(h + 1) * DH] = jnp.dot(
                    w.astype(jnp.bfloat16), v_h,
                    preferred_element_type=jnp.float32).astype(jnp.bfloat16)
            out_ref[b * SQ:(b + 1) * SQ, :] = jnp.dot(
                ctx_ref[:, :], wo_ref[:, :],
                preferred_element_type=jnp.float32)

        seg_start = jnp.int32(0)

        half = ROWS
        for slot, d in enumerate((1, 2, 4, 8)):
            half //= 2
            partner = my_pos ^ d
            has_bit = (my_pos & d) > 0
            send_off = pl.multiple_of(
                jnp.where(has_bit, seg_start, seg_start + half), CHUNK)
            keep_off = pl.multiple_of(
                jnp.where(has_bit, seg_start + half, seg_start), CHUNK)
            sbuf_ref[0:half, :] = out_ref[
                pl.ds(send_off, half), :].astype(jnp.bfloat16)
            rdma = pltpu.make_async_remote_copy(
                src_ref=sbuf_ref.at[0:half, :],
                dst_ref=comm_ref.at[slot, 0:half, :],
                send_sem=send_sems.at[slot],
                recv_sem=recv_sems.at[slot],
                device_id=(partner,),
                device_id_type=pl.DeviceIdType.MESH,
            )
            rdma.start()
            rdma.wait()
            acc = out_ref[pl.ds(keep_off, half), :]
            out_ref[pl.ds(keep_off, half), :] = (
                acc + comm_ref[slot, 0:half, :].astype(jnp.float32))
            seg_start = keep_off

        seg_len = ROWS // N_DEV
        for slot, d in enumerate((8, 4, 2, 1)):
            slot += 4
            partner = my_pos ^ d
            rowbit = (ROWS // 2) // d
            seg_start = pl.multiple_of(seg_start, CHUNK)
            sbuf_ref[0:seg_len, :] = out_ref[
                pl.ds(seg_start, seg_len), :].astype(jnp.bfloat16)
            rdma = pltpu.make_async_remote_copy(
                src_ref=sbuf_ref.at[0:seg_len, :],
                dst_ref=comm_ref.at[slot, 0:seg_len, :],
                send_sem=send_sems.at[slot],
                recv_sem=recv_sems.at[slot],
                device_id=(partner,),
                device_id_type=pl.DeviceIdType.MESH,
            )
            rdma.start()
            rdma.wait()
            partner_start = pl.multiple_of(seg_start ^ rowbit, CHUNK)
            out_ref[pl.ds(partner_start, seg_len), :] = comm_ref[
                slot, 0:seg_len, :].astype(jnp.float32)
            seg_start = jnp.bitwise_and(seg_start, jnp.int32(~rowbit))
            seg_len *= 2

    out = pl.pallas_call(
        body,
        out_shape=jax.ShapeDtypeStruct((ROWS, D_MODEL), jnp.float32),
        in_specs=[pl.BlockSpec(memory_space=pltpu.VMEM)] * 5,
        out_specs=pl.BlockSpec(memory_space=pltpu.VMEM),
        scratch_shapes=[
            pltpu.VMEM((SQ, HQ_PER * DH), jnp.bfloat16),
            pltpu.VMEM((ROWS // 2, D_MODEL), jnp.bfloat16),
            pltpu.VMEM((8, ROWS // 2, D_MODEL), jnp.bfloat16),
            pltpu.SemaphoreType.DMA((8,)),
            pltpu.SemaphoreType.DMA((8,)),
        ],
        compiler_params=pltpu.CompilerParams(collective_id=0),
    )(x, Wq, K_sh, V_sh, Wo)
    return out.reshape(B, SQ, D_MODEL)
